# Initial kernel scaffold; baseline (speedup 1.0000x reference)
import functools

import jax
import jax.numpy as jnp
from jax import lax
from jax.experimental import pallas as pl
from jax.experimental.pallas import tpu as pltpu
from jax.experimental.pallas import tpu_sc as plsc

_NC = 2
_NS = 16
_NW = _NC * _NS


def _make_gather(n_rows, d, b, c_dim, chunk):
    b_per_w = b // _NW
    nch = b_per_w // chunk
    mesh = plsc.VectorSubcoreMesh(
        core_axis_name="c", subcore_axis_name="s", num_cores=_NC,
        num_subcores=_NS)

    @functools.partial(
        pl.kernel,
        mesh=mesh,
        out_type=[
            jax.ShapeDtypeStruct((b, d), jnp.float32),
            jax.ShapeDtypeStruct((b, c_dim), jnp.float32),
        ],
        scratch_types=[
            pltpu.VMEM((b_per_w,), jnp.int32),
            pltpu.VMEM((2, chunk, d), jnp.float32),
            pltpu.VMEM((b_per_w, c_dim), jnp.float32),
            pltpu.SemaphoreType.DMA,
            pltpu.SemaphoreType.DMA,
            pltpu.SemaphoreType.DMA,
            pltpu.SemaphoreType.DMA,
            pltpu.SemaphoreType.DMA,
        ],
    )
    def gather_kernel(spec_hbm, coords_hbm, idx_hbm, out_hbm, lab_hbm,
                      idx_v, rows_v, crows_v, gsem0, gsem1, ssem0, ssem1,
                      csem):
        wid = lax.axis_index("s") * _NC + lax.axis_index("c")
        base = wid * b_per_w

        pltpu.sync_copy(idx_hbm.at[pl.ds(base, b_per_w)], idx_v)

        ccopy = pltpu.async_copy(coords_hbm.at[idx_v], crows_v, csem)

        gsems = (gsem0, gsem1)
        ssems = (ssem0, ssem1)

        def start_gather(c):
            buf = c % 2
            return pltpu.async_copy(
                spec_hbm.at[idx_v.at[pl.ds(c * chunk, chunk)]],
                rows_v.at[buf], gsems[buf])

        gathers = [None, None]
        gathers[0] = start_gather(0)
        if nch > 1:
            gathers[1] = start_gather(1)
        scatters = [None, None]
        for c in range(nch):
            buf = c % 2
            gathers[buf].wait()
            scatters[buf] = pltpu.async_copy(
                rows_v.at[buf],
                out_hbm.at[pl.ds(base + c * chunk, chunk)], ssems[buf])
            if c + 2 < nch:
                scatters[buf].wait()
                gathers[buf] = start_gather(c + 2)
        for c in (nch - 2, nch - 1):
            if c >= 0 and scatters[c % 2] is not None:
                scatters[c % 2].wait()

        ccopy.wait()
        pltpu.sync_copy(crows_v, lab_hbm.at[pl.ds(base, b_per_w)])

    return gather_kernel


def kernel(spectrograms, coords, indices):
    n, h, w = spectrograms.shape
    d = h * w
    b = indices.shape[0]
    c_dim = coords.shape[1]
    spec2d = spectrograms.reshape(n, d)
    samples, labels = _make_gather(n, d, b, c_dim, 8)(
        spec2d, coords, indices)
    return samples.reshape(b, 1, h, w), labels


# speedup vs baseline: 1.2877x; 1.2877x over previous
import functools

import jax
import jax.numpy as jnp
from jax import lax
from jax.experimental import pallas as pl
from jax.experimental.pallas import tpu as pltpu
from jax.experimental.pallas import tpu_sc as plsc

_NC = 2
_NS = 16
_NW = _NC * _NS


def _make_gather(n_rows, d, b, c_dim, chunk):
    b_per_w = b // _NW
    nch = b_per_w // chunk
    mesh = plsc.VectorSubcoreMesh(
        core_axis_name="c", subcore_axis_name="s", num_cores=_NC,
        num_subcores=_NS)

    @functools.partial(
        pl.kernel,
        mesh=mesh,
        out_type=[
            jax.ShapeDtypeStruct((b, d), jnp.float32),
            jax.ShapeDtypeStruct((b, c_dim), jnp.float32),
        ],
        scratch_types=[
            pltpu.VMEM((b_per_w,), jnp.int32),
            pltpu.VMEM((2, chunk, d), jnp.float32),
            pltpu.VMEM((b_per_w, c_dim), jnp.float32),
            pltpu.SemaphoreType.DMA,
            pltpu.SemaphoreType.DMA,
            pltpu.SemaphoreType.DMA,
            pltpu.SemaphoreType.DMA,
            pltpu.SemaphoreType.DMA,
        ],
    )
    def gather_kernel(spec_hbm, coords_hbm, idx_hbm, out_hbm, lab_hbm,
                      idx_v, rows_v, crows_v, gsem0, gsem1, ssem0, ssem1,
                      csem):
        wid = lax.axis_index("s") * _NC + lax.axis_index("c")
        base = wid * b_per_w

        pltpu.sync_copy(idx_hbm.at[pl.ds(base, b_per_w)], idx_v)

        ccopy = pltpu.async_copy(coords_hbm.at[idx_v], crows_v, csem)

        gsems = (gsem0, gsem1)
        ssems = (ssem0, ssem1)

        def start_gather(c):
            buf = c % 2
            return pltpu.async_copy(
                spec_hbm.at[idx_v.at[pl.ds(c * chunk, chunk)]],
                rows_v.at[buf], gsems[buf])

        gathers = [None, None]
        gathers[0] = start_gather(0)
        if nch > 1:
            gathers[1] = start_gather(1)
        scatters = [None, None]
        for c in range(nch):
            buf = c % 2
            gathers[buf].wait()
            scatters[buf] = pltpu.async_copy(
                rows_v.at[buf],
                out_hbm.at[pl.ds(base + c * chunk, chunk)], ssems[buf])
            if c + 2 < nch:
                scatters[buf].wait()
                gathers[buf] = start_gather(c + 2)
        for c in (nch - 2, nch - 1):
            if c >= 0 and scatters[c % 2] is not None:
                scatters[c % 2].wait()

        ccopy.wait()
        pltpu.sync_copy(crows_v, lab_hbm.at[pl.ds(base, b_per_w)])

    return gather_kernel


def kernel(spectrograms, coords, indices):
    n, h, w = spectrograms.shape
    d = h * w
    b = indices.shape[0]
    c_dim = coords.shape[1]
    spec2d = spectrograms.reshape(n, d)
    c_pad = 128
    coords_p = jnp.pad(coords, ((0, 0), (0, c_pad - c_dim)))
    samples, labels = _make_gather(n, d, b, c_pad, 8)(
        spec2d, coords_p, indices)
    return samples.reshape(b, 1, h, w), labels[:, :c_dim]
